# Initial kernel scaffold; baseline (speedup 1.0000x reference)
import jax
import jax.numpy as jnp
from jax.experimental import pallas as pl


def kernel(tec, cal_pos, ant_pos, conv_params, gn_params):
    raise NotImplementedError("write your pallas kernel here")


# speedup vs baseline: 81.1057x; 81.1057x over previous
import jax
import jax.numpy as jnp
from jax.experimental import pallas as pl
from jax.experimental.pallas import tpu as pltpu

B, Nt, Nd, D, CH = 4, 32, 40, 3, 8
H = 16
STEPS = 3
CLASS_BIAS = -2.0
EPS = 1e-5
G = B * Nt
N = G * Nd
NP = N // 8
BG = 16
NBLK = G // BG
BNP = BG * Nd // 8
BR = BG * Nd
BE = BG * Nd * Nd // 8

_f32 = jnp.float32


def _ln(x, mmean, g, b):
    m = jnp.dot(x, mmean, preferred_element_type=_f32)
    d = x - m
    v = jnp.dot(d * d, mmean, preferred_element_type=_f32)
    return g * d * jax.lax.rsqrt(v + EPS) + b


def _ln16(x, g, b):
    m = jnp.mean(x, axis=-1, keepdims=True)
    d = x - m
    v = jnp.mean(d * d, axis=-1, keepdims=True)
    return g * d * jax.lax.rsqrt(v + EPS) + b


def _expand_tile(xp, texp):
    parts = [jnp.dot(xp, texp[j], preferred_element_type=_f32)
             for j in range(8)]
    return jnp.stack(parts, axis=1).reshape(xp.shape[0] * 8, 128)


def _gn_kernel(tec_ref, pos_ref, kPW_ref, pbt_ref, Ks_ref, cbt_ref, at_ref,
               ct_ref, kWn1_ref, kWq_ref, K_ref, V_ref, M_ref, U_ref,
               Tt8_ref, Msn_ref, Texp_ref, kWo_ref, ob_ref, out_ref,
               sn0, sq, spr, sps, spn):
    i = pl.program_id(0)
    mmean = K_ref[15]

    @pl.when(i == 0)
    def _prologue():
        x = jnp.dot(tec_ref[...], kPW_ref[...],
                    preferred_element_type=_f32) + pbt_ref[...]
        for blk in range(4):
            x4 = x.reshape(B, Nt, Nd // 8, 64)
            z = jnp.zeros((B, 1, Nd // 8, 64), _f32)
            xm = jnp.concatenate([z, x4[:, :-1]], axis=1).reshape(NP, 64)
            xp = jnp.concatenate([x4[:, 1:], z], axis=1).reshape(NP, 64)
            h = (jnp.dot(xm, Ks_ref[blk, 0], preferred_element_type=_f32)
                 + jnp.dot(x, Ks_ref[blk, 1], preferred_element_type=_f32)
                 + jnp.dot(xp, Ks_ref[blk, 2], preferred_element_type=_f32)
                 + cbt_ref[blk])
            h = jnp.maximum(h, 0.0)
            x = at_ref[blk] * h + ct_ref[blk] + x
        l1 = jnp.maximum(jnp.dot(x, kWn1_ref[...],
                                 preferred_element_type=_f32) + V_ref[4], 0.0)
        l2 = jnp.maximum(jnp.dot(l1, K_ref[12],
                                 preferred_element_type=_f32) + V_ref[5], 0.0)
        n0 = _ln(l2, mmean, V_ref[6], V_ref[7])
        sn0[...] = n0
        sq[...] = jnp.dot(pos_ref[...], kWq_ref[...],
                          preferred_element_type=_f32)
        spr[...] = jnp.dot(n0, K_ref[2], preferred_element_type=_f32)
        sps[...] = jnp.dot(n0, K_ref[4], preferred_element_type=_f32)
        spn[...] = jnp.dot(n0, K_ref[6], preferred_element_type=_f32)

    r0 = i * BNP
    n0p = sn0[pl.ds(r0, BNP), :]
    qp = sq[pl.ds(r0, BNP), :]
    pr_b = spr[pl.ds(r0, BNP), :]
    ps_b = sps[pl.ds(r0, BNP), :]
    pn_b = spn[pl.ds(r0, BNP), :]
    tt8 = Tt8_ref[...]
    msn = Msn_ref[...]
    texp = Texp_ref[...]

    v1 = jnp.maximum(U_ref[6], 0.0).reshape(1, 16)
    v2 = jnp.maximum(jnp.dot(v1, M_ref[8], preferred_element_type=_f32)
                     + U_ref[7], 0.0)
    g0c = _ln16(v2, U_ref[8], U_ref[9])

    qr = jnp.broadcast_to(qp.reshape(BNP, 1, 128), (BNP, Nd, 128)).reshape(BE, 128)
    qs3 = _expand_tile(qp, texp).reshape(BG, Nd, 128)
    qs = jnp.tile(qs3, (1, Nd // 8, 1)).reshape(BE, 128)
    h = jnp.maximum(qr - qs + V_ref[0], 0.0)
    h2 = jnp.maximum(jnp.dot(h, K_ref[9], preferred_element_type=_f32)
                     + V_ref[1], 0.0)
    e = _ln(h2, mmean, V_ref[2], V_ref[3])
    pre_e0 = jnp.dot(e, K_ref[0], preferred_element_type=_f32)

    np_ = n0p
    gst = jnp.broadcast_to(g0c, (BG, 16))
    tg0 = jnp.dot(g0c, M_ref[0], preferred_element_type=_f32)
    ug0 = jnp.dot(g0c, M_ref[2], preferred_element_type=_f32)
    cg0 = jnp.dot(g0c, M_ref[6], preferred_element_type=_f32)

    for _ in range(STEPS):
        t_r = pr_b + jnp.dot(np_, K_ref[3], preferred_element_type=_f32)
        t_rb = jnp.broadcast_to(t_r.reshape(BNP, 1, 128),
                                (BNP, Nd, 128)).reshape(BE, 128)
        w = ps_b + jnp.dot(np_, K_ref[5], preferred_element_type=_f32)
        ts3 = _expand_tile(w, texp).reshape(BG, Nd, 128)
        tsb = jnp.tile(ts3, (1, Nd // 8, 1)).reshape(BE, 128)
        tg = tg0 + jnp.dot(gst, M_ref[1], preferred_element_type=_f32) + U_ref[0]
        tgb = jnp.broadcast_to(
            jnp.dot(tg, tt8, preferred_element_type=_f32).reshape(BG, 1, 128),
            (BG, Nd * Nd // 8, 128)).reshape(BE, 128)
        eA2 = jnp.dot(e, K_ref[1], preferred_element_type=_f32)
        h = jnp.maximum(pre_e0 + eA2 + t_rb + tsb + tgb, 0.0)
        h2 = jnp.maximum(jnp.dot(h, K_ref[10], preferred_element_type=_f32)
                         + V_ref[8], 0.0)
        e_new = _ln(h2, mmean, V_ref[9], V_ref[10])

        agg = jnp.sum(e_new.reshape(BNP, Nd, 128), axis=1)

        ug = ug0 + jnp.dot(gst, M_ref[3], preferred_element_type=_f32) + U_ref[1]
        ugb = jnp.broadcast_to(
            jnp.dot(ug, tt8, preferred_element_type=_f32).reshape(BG, 1, 128),
            (BG, Nd // 8, 128)).reshape(BNP, 128)
        hn = jnp.maximum(jnp.dot(agg, K_ref[7], preferred_element_type=_f32)
                         + pn_b
                         + jnp.dot(np_, K_ref[8], preferred_element_type=_f32)
                         + ugb, 0.0)
        hn2 = jnp.maximum(jnp.dot(hn, K_ref[11], preferred_element_type=_f32)
                          + V_ref[11], 0.0)
        n_new = _ln(hn2, mmean, V_ref[12], V_ref[13])

        ge = jnp.dot(jnp.sum(agg.reshape(BG, Nd // 8, 128), axis=1), msn,
                     preferred_element_type=_f32)
        gn = jnp.dot(jnp.sum(n_new.reshape(BG, Nd // 8, 128), axis=1), msn,
                     preferred_element_type=_f32)
        hg = jnp.maximum(jnp.dot(ge, M_ref[4], preferred_element_type=_f32)
                         + jnp.dot(gn, M_ref[5], preferred_element_type=_f32)
                         + cg0
                         + jnp.dot(gst, M_ref[7], preferred_element_type=_f32)
                         + U_ref[2], 0.0)
        hg2 = jnp.maximum(jnp.dot(hg, M_ref[9], preferred_element_type=_f32)
                          + U_ref[3], 0.0)
        g_new = _ln16(hg2, U_ref[4], U_ref[5])

        np_ = n_new + np_
        e = e_new
        gst = g_new

    d1 = jnp.maximum(jnp.dot(np_, K_ref[13], preferred_element_type=_f32)
                     + V_ref[14], 0.0)
    d2 = jnp.maximum(jnp.dot(d1, K_ref[14], preferred_element_type=_f32)
                     + V_ref[15], 0.0)
    dn = _ln(d2, mmean, V_ref[16], V_ref[17])
    out_ref[...] = jnp.dot(dn, kWo_ref[...],
                           preferred_element_type=_f32) + ob_ref[...]


def kernel(tec, cal_pos, ant_pos, conv_params, gn_params):
    pw, pb, blocks = conv_params
    (enc_e, enc_n, enc_g, core_e, core_n, core_g,
     dec_e, dec_n, dec_g, out_n) = gn_params
    (We1, be1), (We2, be2) = enc_e[0]; ge_, bee = enc_e[1]
    (Wn1, bn1), (Wn2, bn2) = enc_n[0]; gn_, bnn = enc_n[1]
    (Wg1, bg1), (Wg2, bg2) = enc_g[0]; gg_, bgg = enc_g[1]
    (Wce1, bce1), (Wce2, bce2) = core_e[0]; gce, bce = core_e[1]
    (Wcn1, bcn1), (Wcn2, bcn2) = core_n[0]; gcn, bcn = core_n[1]
    (Wcg1, bcg1), (Wcg2, bcg2) = core_g[0]; gcg, bcg = core_g[1]
    (Wdn1, bdn1), (Wdn2, bdn2) = dec_n[0]; gdn, bdn = dec_n[1]
    Wo, bo = out_n

    I8 = jnp.eye(8, dtype=_f32)
    I16 = jnp.eye(16, dtype=_f32)
    I40 = jnp.eye(Nd, dtype=_f32)
    kron = jnp.kron
    t8 = lambda v: jnp.tile(v, 8)

    kPW = kron(I8, pw.reshape(1, CH))
    pbt = jnp.tile(pb, 8).reshape(1, 64)
    Ks = jnp.stack([jnp.stack([kron(I8, cw[k, 0]) for k in range(3)])
                    for (cw, _, _, _, _, _) in blocks])
    cbt = jnp.stack([jnp.tile(cb, 8) for (_, cb, _, _, _, _) in blocks])
    a_l, c_l = [], []
    for (_, _, gma, bta, mu, var) in blocks:
        a = gma * jax.lax.rsqrt(var + 1e-3)
        a_l.append(jnp.tile(a, 8))
        c_l.append(jnp.tile(bta - a * mu, 8))
    at = jnp.stack(a_l)
    ct = jnp.stack(c_l)

    K128 = jnp.stack([
        kron(I8, Wce1[0:16]),
        kron(I8, Wce1[16:32]),
        kron(I8, Wce1[32:48]),
        kron(I8, Wce1[48:64]),
        kron(I8, Wce1[64:80]),
        kron(I8, Wce1[80:96]),
        kron(I8, Wcn1[16:32]),
        kron(I8, Wcn1[0:16]),
        kron(I8, Wcn1[32:48]),
        kron(I8, We2),
        kron(I8, Wce2),
        kron(I8, Wcn2),
        kron(I8, Wn2),
        kron(I8, Wdn1),
        kron(I8, Wdn2),
        kron(I8, jnp.ones((16, 16), _f32) / 16.0),
    ])
    V128 = jnp.stack([
        t8(be1), t8(be2), t8(ge_), t8(bee),
        t8(bn1), t8(bn2), t8(gn_), t8(bnn),
        t8(bce2), t8(gce), t8(bce),
        t8(bcn2), t8(gcn), t8(bcn),
        t8(bdn1), t8(bdn2), t8(gdn), t8(bdn),
    ])
    M16 = jnp.stack([
        Wce1[96:112], Wce1[112:128],
        Wcn1[48:64], Wcn1[64:80],
        Wcg1[0:16], Wcg1[16:32],
        Wcg1[32:48], Wcg1[48:64],
        Wg2, Wcg2,
    ])
    V16 = jnp.stack([bce1, bcn1, bcg1, bcg2, gcg, bcg,
                     bg1, bg2, gg_, bgg])
    kWn1 = kron(I8, Wn1)
    kWq = kron(I8, We1)
    Tt8 = kron(jnp.ones((1, 8), _f32), I16)
    Msn = kron(jnp.ones((8, 1), _f32), I16)
    eye8 = jnp.eye(8, dtype=_f32)
    Texp = jnp.stack([kron(jnp.outer(eye8[j], jnp.ones((8,), _f32)), I16)
                      for j in range(8)])
    kWo = kron(I8, Wo)
    ob = (jnp.tile(bo, 8) + CLASS_BIAS).reshape(1, 8)

    tecp = tec.reshape(NP, 8)
    pos2 = cal_pos.reshape(NP, 8 * D)

    args = (tecp, pos2, kPW, pbt, Ks, cbt, at, ct, kWn1, kWq,
            K128, V128, M16, V16, Tt8, Msn, Texp, kWo, ob)

    def full(a):
        nd = a.ndim
        return pl.BlockSpec(a.shape, lambda i, _n=nd: (0,) * _n)

    out = pl.pallas_call(
        _gn_kernel,
        grid=(NBLK,),
        in_specs=[full(a) for a in args],
        out_specs=pl.BlockSpec((BNP, 8), lambda i: (i, 0)),
        out_shape=jax.ShapeDtypeStruct((NP, 8), _f32),
        scratch_shapes=[pltpu.VMEM((NP, 128), _f32) for _ in range(5)],
        compiler_params=pltpu.CompilerParams(
            dimension_semantics=("arbitrary",)),
    )(*args)
    return out.reshape(B, Nt, Nd, 1)


# speedup vs baseline: 84.0779x; 1.0366x over previous
import jax
import jax.numpy as jnp
from jax.experimental import pallas as pl
from jax.experimental.pallas import tpu as pltpu

B, Nt, Nd, D, CH = 4, 32, 40, 3, 8
H = 16
STEPS = 3
CLASS_BIAS = -2.0
EPS = 1e-5
G = B * Nt
N = G * Nd
NP = N // 8
BG = 16
NBLK = G // BG
BNP = BG * Nd // 8
BR = BG * Nd
BE = BG * Nd * Nd // 8

_f32 = jnp.float32


def _ln(x, mmean, g, b):
    m = jnp.dot(x, mmean, preferred_element_type=_f32)
    d = x - m
    v = jnp.dot(d * d, mmean, preferred_element_type=_f32)
    return g * d * jax.lax.rsqrt(v + EPS) + b


def _ln16(x, g, b):
    m = jnp.mean(x, axis=-1, keepdims=True)
    d = x - m
    v = jnp.mean(d * d, axis=-1, keepdims=True)
    return g * d * jax.lax.rsqrt(v + EPS) + b


def _expand_tile(xp, mask, tones):
    em = (xp[:, None, :] * mask).reshape(xp.shape[0] * 8, 128)
    return jnp.dot(em, tones, preferred_element_type=_f32)


def _gn_kernel(tec_ref, pos_ref, kPW_ref, pbt_ref, Ks_ref, cbt_ref, at_ref,
               ct_ref, kWn1_ref, kWq_ref, K_ref, V_ref, M_ref, U_ref,
               Tt8_ref, Msn_ref, Emask_ref, Tones_ref, kWo_ref, ob_ref,
               out_ref,
               sn0, sq, spr, sps, spn):
    i = pl.program_id(0)
    mmean = K_ref[15]

    @pl.when(i == 0)
    def _prologue():
        x = jnp.dot(tec_ref[...], kPW_ref[...],
                    preferred_element_type=_f32) + pbt_ref[...]
        for blk in range(4):
            x4 = x.reshape(B, Nt, Nd // 8, 64)
            z = jnp.zeros((B, 1, Nd // 8, 64), _f32)
            xm = jnp.concatenate([z, x4[:, :-1]], axis=1).reshape(NP, 64)
            xp = jnp.concatenate([x4[:, 1:], z], axis=1).reshape(NP, 64)
            h = (jnp.dot(xm, Ks_ref[blk, 0], preferred_element_type=_f32)
                 + jnp.dot(x, Ks_ref[blk, 1], preferred_element_type=_f32)
                 + jnp.dot(xp, Ks_ref[blk, 2], preferred_element_type=_f32)
                 + cbt_ref[blk])
            h = jnp.maximum(h, 0.0)
            x = at_ref[blk] * h + ct_ref[blk] + x
        l1 = jnp.maximum(jnp.dot(x, kWn1_ref[...],
                                 preferred_element_type=_f32) + V_ref[4], 0.0)
        l2 = jnp.maximum(jnp.dot(l1, K_ref[12],
                                 preferred_element_type=_f32) + V_ref[5], 0.0)
        n0 = _ln(l2, mmean, V_ref[6], V_ref[7])
        sn0[...] = n0
        sq[...] = jnp.dot(pos_ref[...], kWq_ref[...],
                          preferred_element_type=_f32)
        spr[...] = jnp.dot(n0, K_ref[2], preferred_element_type=_f32)
        sps[...] = jnp.dot(n0, K_ref[4], preferred_element_type=_f32)
        spn[...] = jnp.dot(n0, K_ref[6], preferred_element_type=_f32)

    r0 = i * BNP
    n0p = sn0[pl.ds(r0, BNP), :]
    qp = sq[pl.ds(r0, BNP), :]
    pr_b = spr[pl.ds(r0, BNP), :]
    ps_b = sps[pl.ds(r0, BNP), :]
    pn_b = spn[pl.ds(r0, BNP), :]
    tt8 = Tt8_ref[...]
    msn = Msn_ref[...]
    emask = Emask_ref[...]
    tones = Tones_ref[...]

    v1 = jnp.maximum(U_ref[6], 0.0).reshape(1, 16)
    v2 = jnp.maximum(jnp.dot(v1, M_ref[8], preferred_element_type=_f32)
                     + U_ref[7], 0.0)
    g0c = _ln16(v2, U_ref[8], U_ref[9])

    q4r = (qp + V_ref[0]).reshape(BG, Nd // 8, 1, 128)
    qs4 = _expand_tile(qp, emask, tones).reshape(BG, 1, Nd, 128)
    h = jnp.maximum(q4r - qs4, 0.0).reshape(BE, 128)
    h2 = jnp.maximum(jnp.dot(h, K_ref[9], preferred_element_type=_f32)
                     + V_ref[1], 0.0)
    e = _ln(h2, mmean, V_ref[2], V_ref[3])
    pre_e0 = jnp.dot(e, K_ref[0], preferred_element_type=_f32)

    np_ = n0p
    gst = jnp.broadcast_to(g0c, (BG, 16))
    tg0 = jnp.dot(g0c, M_ref[0], preferred_element_type=_f32)
    ug0 = jnp.dot(g0c, M_ref[2], preferred_element_type=_f32)
    cg0 = jnp.dot(g0c, M_ref[6], preferred_element_type=_f32)

    for _ in range(STEPS):
        t_r = pr_b + jnp.dot(np_, K_ref[3], preferred_element_type=_f32)
        tg = tg0 + jnp.dot(gst, M_ref[1], preferred_element_type=_f32) + U_ref[0]
        tgt = jnp.dot(tg, tt8, preferred_element_type=_f32)
        w = ps_b + jnp.dot(np_, K_ref[5], preferred_element_type=_f32)
        w = (w.reshape(BG, Nd // 8, 128) + tgt[:, None, :]).reshape(BNP, 128)
        ts4 = _expand_tile(w, emask, tones).reshape(BG, 1, Nd, 128)
        epre = pre_e0 + jnp.dot(e, K_ref[1], preferred_element_type=_f32)
        h = jnp.maximum(epre.reshape(BG, Nd // 8, Nd, 128)
                        + t_r.reshape(BG, Nd // 8, 1, 128) + ts4,
                        0.0).reshape(BE, 128)
        h2 = jnp.maximum(jnp.dot(h, K_ref[10], preferred_element_type=_f32)
                         + V_ref[8], 0.0)
        e_new = _ln(h2, mmean, V_ref[9], V_ref[10])

        agg = jnp.sum(e_new.reshape(BNP, Nd, 128), axis=1)

        ug = ug0 + jnp.dot(gst, M_ref[3], preferred_element_type=_f32) + U_ref[1]
        ugt = jnp.dot(ug, tt8, preferred_element_type=_f32)
        hn0 = (jnp.dot(agg, K_ref[7], preferred_element_type=_f32)
               + pn_b
               + jnp.dot(np_, K_ref[8], preferred_element_type=_f32))
        hn = jnp.maximum((hn0.reshape(BG, Nd // 8, 128)
                          + ugt[:, None, :]).reshape(BNP, 128), 0.0)
        hn2 = jnp.maximum(jnp.dot(hn, K_ref[11], preferred_element_type=_f32)
                          + V_ref[11], 0.0)
        n_new = _ln(hn2, mmean, V_ref[12], V_ref[13])

        ge = jnp.dot(jnp.sum(agg.reshape(BG, Nd // 8, 128), axis=1), msn,
                     preferred_element_type=_f32)
        gn = jnp.dot(jnp.sum(n_new.reshape(BG, Nd // 8, 128), axis=1), msn,
                     preferred_element_type=_f32)
        hg = jnp.maximum(jnp.dot(ge, M_ref[4], preferred_element_type=_f32)
                         + jnp.dot(gn, M_ref[5], preferred_element_type=_f32)
                         + cg0
                         + jnp.dot(gst, M_ref[7], preferred_element_type=_f32)
                         + U_ref[2], 0.0)
        hg2 = jnp.maximum(jnp.dot(hg, M_ref[9], preferred_element_type=_f32)
                          + U_ref[3], 0.0)
        g_new = _ln16(hg2, U_ref[4], U_ref[5])

        np_ = n_new + np_
        e = e_new
        gst = g_new

    d1 = jnp.maximum(jnp.dot(np_, K_ref[13], preferred_element_type=_f32)
                     + V_ref[14], 0.0)
    d2 = jnp.maximum(jnp.dot(d1, K_ref[14], preferred_element_type=_f32)
                     + V_ref[15], 0.0)
    dn = _ln(d2, mmean, V_ref[16], V_ref[17])
    out_ref[...] = jnp.dot(dn, kWo_ref[...],
                           preferred_element_type=_f32) + ob_ref[...]


def kernel(tec, cal_pos, ant_pos, conv_params, gn_params):
    pw, pb, blocks = conv_params
    (enc_e, enc_n, enc_g, core_e, core_n, core_g,
     dec_e, dec_n, dec_g, out_n) = gn_params
    (We1, be1), (We2, be2) = enc_e[0]; ge_, bee = enc_e[1]
    (Wn1, bn1), (Wn2, bn2) = enc_n[0]; gn_, bnn = enc_n[1]
    (Wg1, bg1), (Wg2, bg2) = enc_g[0]; gg_, bgg = enc_g[1]
    (Wce1, bce1), (Wce2, bce2) = core_e[0]; gce, bce = core_e[1]
    (Wcn1, bcn1), (Wcn2, bcn2) = core_n[0]; gcn, bcn = core_n[1]
    (Wcg1, bcg1), (Wcg2, bcg2) = core_g[0]; gcg, bcg = core_g[1]
    (Wdn1, bdn1), (Wdn2, bdn2) = dec_n[0]; gdn, bdn = dec_n[1]
    Wo, bo = out_n

    I8 = jnp.eye(8, dtype=_f32)
    I16 = jnp.eye(16, dtype=_f32)
    I40 = jnp.eye(Nd, dtype=_f32)
    kron = jnp.kron
    t8 = lambda v: jnp.tile(v, 8)

    kPW = kron(I8, pw.reshape(1, CH))
    pbt = jnp.tile(pb, 8).reshape(1, 64)
    Ks = jnp.stack([jnp.stack([kron(I8, cw[k, 0]) for k in range(3)])
                    for (cw, _, _, _, _, _) in blocks])
    cbt = jnp.stack([jnp.tile(cb, 8) for (_, cb, _, _, _, _) in blocks])
    a_l, c_l = [], []
    for (_, _, gma, bta, mu, var) in blocks:
        a = gma * jax.lax.rsqrt(var + 1e-3)
        a_l.append(jnp.tile(a, 8))
        c_l.append(jnp.tile(bta - a * mu, 8))
    at = jnp.stack(a_l)
    ct = jnp.stack(c_l)

    K128 = jnp.stack([
        kron(I8, Wce1[0:16]),
        kron(I8, Wce1[16:32]),
        kron(I8, Wce1[32:48]),
        kron(I8, Wce1[48:64]),
        kron(I8, Wce1[64:80]),
        kron(I8, Wce1[80:96]),
        kron(I8, Wcn1[16:32]),
        kron(I8, Wcn1[0:16]),
        kron(I8, Wcn1[32:48]),
        kron(I8, We2),
        kron(I8, Wce2),
        kron(I8, Wcn2),
        kron(I8, Wn2),
        kron(I8, Wdn1),
        kron(I8, Wdn2),
        kron(I8, jnp.ones((16, 16), _f32) / 16.0),
    ])
    V128 = jnp.stack([
        t8(be1), t8(be2), t8(ge_), t8(bee),
        t8(bn1), t8(bn2), t8(gn_), t8(bnn),
        t8(bce2), t8(gce), t8(bce),
        t8(bcn2), t8(gcn), t8(bcn),
        t8(bdn1), t8(bdn2), t8(gdn), t8(bdn),
    ])
    M16 = jnp.stack([
        Wce1[96:112], Wce1[112:128],
        Wcn1[48:64], Wcn1[64:80],
        Wcg1[0:16], Wcg1[16:32],
        Wcg1[32:48], Wcg1[48:64],
        Wg2, Wcg2,
    ])
    V16 = jnp.stack([bce1, bcn1, bcg1, bcg2, gcg, bcg,
                     bg1, bg2, gg_, bgg])
    kWn1 = kron(I8, Wn1)
    kWq = kron(I8, We1)
    Tt8 = kron(jnp.ones((1, 8), _f32), I16)
    Msn = kron(jnp.ones((8, 1), _f32), I16)
    Emask = kron(jnp.eye(8, dtype=_f32), jnp.ones((1, 16), _f32))
    Tones = kron(jnp.ones((8, 8), _f32), I16)
    kWo = kron(I8, Wo)
    ob = (jnp.tile(bo, 8) + CLASS_BIAS).reshape(1, 8)

    tecp = tec.reshape(NP, 8)
    pos2 = cal_pos.reshape(NP, 8 * D)

    args = (tecp, pos2, kPW, pbt, Ks, cbt, at, ct, kWn1, kWq,
            K128, V128, M16, V16, Tt8, Msn, Emask, Tones, kWo, ob)

    def full(a):
        nd = a.ndim
        return pl.BlockSpec(a.shape, lambda i, _n=nd: (0,) * _n)

    out = pl.pallas_call(
        _gn_kernel,
        grid=(NBLK,),
        in_specs=[full(a) for a in args],
        out_specs=pl.BlockSpec((BNP, 8), lambda i: (i, 0)),
        out_shape=jax.ShapeDtypeStruct((NP, 8), _f32),
        scratch_shapes=[pltpu.VMEM((NP, 128), _f32) for _ in range(5)],
        compiler_params=pltpu.CompilerParams(
            dimension_semantics=("arbitrary",)),
    )(*args)
    return out.reshape(B, Nt, Nd, 1)
